# baseline (speedup 1.0000x reference)
import functools

import jax
import jax.numpy as jnp
from jax import lax
from jax.experimental import pallas as pl
from jax.experimental.pallas import tpu as pltpu
from jax.experimental.pallas import tpu_sc as plsc

VOCAB = 100000
PAD = 1
L = 200
B = 1024

NC, NS, LANES = 2, 16, 16
NW = NC * NS
SENT_PER_W = B // NW
LPAD = 208
CHUNKS = LPAD // LANES
TOK_PER_W = SENT_PER_W * LPAD
GSPLIT = 8
SENT_PER_G = SENT_PER_W // GSPLIT
TOK_PER_G = SENT_PER_G * LPAD


def _nb_body(sent_hbm, lcr_hbm, out_hbm, toks2_v, toks_v, vals_v, stamp_v,
             score_v, lcr_sh, sem_t, sem_tbl, sem_g):
    cid = lax.axis_index("c")
    sid = lax.axis_index("s")
    wid = sid * NC + cid

    with jax.named_scope("stage_tokens_start"):
        tok_copy = pltpu.async_copy(
            sent_hbm.at[pl.ds(wid * SENT_PER_W, SENT_PER_W), :], toks2_v,
            sem_t)

    with jax.named_scope("stage_table_start"):
        @pl.when(sid == 0)
        def _():
            pltpu.async_copy(lcr_hbm, lcr_sh, sem_tbl)

    with jax.named_scope("stage_tokens_wait"):
        tok_copy.wait()

    with jax.named_scope("flatten_tokens"):
        def flatten(s, carry):
            for k in range(CHUNKS):
                toks_v[pl.ds(s * LPAD + k * LANES, LANES)] = (
                    toks2_v[s, pl.ds(k * LANES, LANES)])
            return carry

        lax.fori_loop(0, SENT_PER_W, flatten, 0)

    with jax.named_scope("stage_table_wait"):
        @pl.when(sid == 0)
        def _():
            pltpu.make_async_copy(lcr_hbm, lcr_sh, sem_tbl).wait()

        plsc.subcore_barrier()

    with jax.named_scope("gather_fire"):
        gathers = [
            pltpu.async_copy(
                lcr_sh.at[toks_v.at[pl.ds(g * TOK_PER_G, TOK_PER_G)]],
                vals_v.at[pl.ds(g * TOK_PER_G, TOK_PER_G)], sem_g)
            for g in range(GSPLIT)
        ]

    lanes = lax.iota(jnp.int32, LANES)

    def sentence(s, carry):
        base = s * LPAD
        for k in range(CHUNKS):
            tok = toks_v[pl.ds(base + k * LANES, LANES)]
            marker = lanes + (s * 256 + k * LANES)
            plsc.store_scatter(stamp_v, [tok], marker)
        acc = jnp.zeros((LANES,), jnp.float32)
        for k in range(CHUNKS):
            tok = toks_v[pl.ds(base + k * LANES, LANES)]
            val = vals_v[pl.ds(base + k * LANES, LANES)]
            back = plsc.load_gather(stamp_v, [tok])
            marker = lanes + (s * 256 + k * LANES)
            keep = (back == marker) & (tok != PAD)
            acc = acc + jnp.where(keep, val, 0.0)
        total = jnp.sum(acc)
        plsc.store_scatter(
            score_v,
            [jnp.zeros((LANES,), jnp.int32) + s],
            jnp.broadcast_to(total, (LANES,)),
            mask=lanes == 0,
        )
        return carry

    for g in range(GSPLIT):
        with jax.named_scope("gather_wait"):
            gathers[g].wait()
        with jax.named_scope("dedup_compute"):
            lax.fori_loop(g * SENT_PER_G, (g + 1) * SENT_PER_G, sentence, 0)

    with jax.named_scope("store_scores"):
        pltpu.sync_copy(score_v, out_hbm.at[pl.ds(wid * SENT_PER_W, SENT_PER_W)])


_nb_kernel = functools.partial(
    pl.kernel,
    out_type=jax.ShapeDtypeStruct((B,), jnp.float32),
    mesh=plsc.VectorSubcoreMesh(core_axis_name="c", subcore_axis_name="s"),
    compiler_params=pltpu.CompilerParams(needs_layout_passes=False),
    scratch_types=[
        pltpu.VMEM((SENT_PER_W, LPAD), jnp.int32),
        pltpu.VMEM((TOK_PER_W,), jnp.int32),
        pltpu.VMEM((TOK_PER_W,), jnp.float32),
        pltpu.VMEM((VOCAB,), jnp.int32),
        pltpu.VMEM((SENT_PER_W,), jnp.float32),
        pltpu.VMEM_SHARED((VOCAB,), jnp.float32),
        pltpu.SemaphoreType.DMA,
        pltpu.SemaphoreType.DMA,
        pltpu.SemaphoreType.DMA,
    ],
)(_nb_body)


@jax.jit
def kernel(sentences, log_count_ratio, bias):
    sent_t = jnp.pad(sentences, ((0, LPAD - L), (0, 0)), constant_values=PAD).T
    scores = _nb_kernel(sent_t, log_count_ratio) + bias
    return jnp.stack([-scores, scores], axis=1)


# speedup vs baseline: 1.0707x; 1.0707x over previous
import functools

import jax
import jax.numpy as jnp
from jax import lax
from jax.experimental import pallas as pl
from jax.experimental.pallas import tpu as pltpu
from jax.experimental.pallas import tpu_sc as plsc

VOCAB = 100000
PAD = 1
L = 200
B = 1024

NC, NS, LANES = 2, 16, 16
NW = NC * NS
SENT_PER_W = B // NW
CHUNKS = (L + LANES - 1) // LANES
TAIL = L - (CHUNKS - 1) * LANES
TOK_PER_W = SENT_PER_W * L
GSPLIT = 8
SENT_PER_G = SENT_PER_W // GSPLIT
TOK_PER_G = SENT_PER_G * L


def _nb_body(toks_hbm, lcr_hbm, out_hbm, toks_v, vals_v, stamp_v, score_v,
             scoret_v, lcr_sh, sem_t, sem_g):
    cid = lax.axis_index("c")
    sid = lax.axis_index("s")
    wid = sid * NC + cid

    with jax.named_scope("stage_tokens_start"):
        tok_copy = pltpu.async_copy(
            toks_hbm.at[pl.ds(wid * TOK_PER_W, TOK_PER_W)],
            toks_v.at[pl.ds(0, TOK_PER_W)], sem_t)

    with jax.named_scope("stage_table"):
        @pl.when(sid == 0)
        def _():
            pltpu.sync_copy(lcr_hbm, lcr_sh)

        plsc.subcore_barrier()

    with jax.named_scope("stage_tokens_wait"):
        tok_copy.wait()

    with jax.named_scope("gather_fire"):
        gathers = []
        for g in range(GSPLIT):
            gathers.append(pltpu.async_copy(
                lcr_sh.at[toks_v.at[pl.ds(g * TOK_PER_G, TOK_PER_G)]],
                vals_v.at[pl.ds(g * TOK_PER_G, TOK_PER_G)], sem_g))

    lanes = lax.iota(jnp.int32, LANES)
    tail_mask = lanes < TAIL
    lane_base = lanes * SENT_PER_W

    def sentence(s, carry):
        base = s * L
        for k in range(CHUNKS):
            tok = toks_v[pl.ds(base + k * LANES, LANES)]
            marker = lanes + (s * 256 + k * LANES)
            if k == CHUNKS - 1:
                plsc.store_scatter(stamp_v, [tok], marker, mask=tail_mask)
            else:
                plsc.store_scatter(stamp_v, [tok], marker)
        acc = jnp.zeros((LANES,), jnp.float32)
        for k in range(CHUNKS):
            tok = toks_v[pl.ds(base + k * LANES, LANES)]
            val = vals_v[pl.ds(base + k * LANES, LANES)]
            if k == CHUNKS - 1:
                back = plsc.load_gather(stamp_v, [tok], mask=tail_mask)
            else:
                back = plsc.load_gather(stamp_v, [tok])
            marker = lanes + (s * 256 + k * LANES)
            keep = (back == marker) & (tok != PAD)
            if k == CHUNKS - 1:
                keep = keep & tail_mask
            acc = acc + jnp.where(keep, val, 0.0)
        plsc.store_scatter(scoret_v, [lane_base + s], acc)
        return carry

    for g in range(GSPLIT):
        with jax.named_scope("gather_wait"):
            gathers[g].wait()
        with jax.named_scope("dedup_compute"):
            lax.fori_loop(g * SENT_PER_G, (g + 1) * SENT_PER_G, sentence, 0)

    with jax.named_scope("score_fold"):
        half0 = jnp.zeros((LANES,), jnp.float32)
        half1 = jnp.zeros((LANES,), jnp.float32)
        for r in range(LANES):
            half0 = half0 + scoret_v[pl.ds(r * SENT_PER_W, LANES)]
            half1 = half1 + scoret_v[pl.ds(r * SENT_PER_W + LANES, LANES)]
        score_v[pl.ds(0, LANES)] = half0
        score_v[pl.ds(LANES, LANES)] = half1

    with jax.named_scope("store_scores"):
        pltpu.sync_copy(score_v, out_hbm.at[pl.ds(wid * SENT_PER_W, SENT_PER_W)])


_nb_kernel = functools.partial(
    pl.kernel,
    out_type=jax.ShapeDtypeStruct((B,), jnp.float32),
    mesh=plsc.VectorSubcoreMesh(core_axis_name="c", subcore_axis_name="s"),
    compiler_params=pltpu.CompilerParams(needs_layout_passes=False),
    scratch_types=[
        pltpu.VMEM((TOK_PER_W + LANES,), jnp.int32),
        pltpu.VMEM((TOK_PER_W + LANES,), jnp.float32),
        pltpu.VMEM((VOCAB,), jnp.int32),
        pltpu.VMEM((SENT_PER_W,), jnp.float32),
        pltpu.VMEM((LANES * SENT_PER_W,), jnp.float32),
        pltpu.VMEM_SHARED((VOCAB,), jnp.float32),
        pltpu.SemaphoreType.DMA,
        pltpu.SemaphoreType.DMA,
    ],
)(_nb_body)


@jax.jit
def kernel(sentences, log_count_ratio, bias):
    toks = sentences.T.reshape(B * L)
    scores = _nb_kernel(toks, log_count_ratio) + bias
    return jnp.stack([-scores, scores], axis=1)


# speedup vs baseline: 1.0827x; 1.0112x over previous
import functools

import jax
import jax.numpy as jnp
from jax import lax
from jax.experimental import pallas as pl
from jax.experimental.pallas import tpu as pltpu
from jax.experimental.pallas import tpu_sc as plsc

VOCAB = 100000
PAD = 1
L = 200
B = 1024

NC, NS, LANES = 2, 16, 16
NW = NC * NS
SENT_PER_W = B // NW
CHUNKS = (L + LANES - 1) // LANES
TAIL = L - (CHUNKS - 1) * LANES
TOK_PER_W = SENT_PER_W * L
GSPLIT = 4
SENT_PER_G = SENT_PER_W // GSPLIT
TOK_PER_G = SENT_PER_G * L


def _nb_body(toks_hbm, lcr_hbm, out_hbm, toks_v, vals_v, stamp_v, score_v,
             scoret_v, lcr_sh, sem_t, sem_g):
    cid = lax.axis_index("c")
    sid = lax.axis_index("s")
    wid = sid * NC + cid

    with jax.named_scope("stage_tokens_start"):
        tok_copy = pltpu.async_copy(
            toks_hbm.at[pl.ds(wid * TOK_PER_W, TOK_PER_W)],
            toks_v.at[pl.ds(0, TOK_PER_W)], sem_t)

    with jax.named_scope("stage_table"):
        @pl.when(sid == 0)
        def _():
            pltpu.sync_copy(lcr_hbm, lcr_sh)

        plsc.subcore_barrier()

    with jax.named_scope("stage_tokens_wait"):
        tok_copy.wait()

    with jax.named_scope("gather_fire"):
        gathers = []
        for g in range(GSPLIT):
            gathers.append(pltpu.async_copy(
                lcr_sh.at[toks_v.at[pl.ds(g * TOK_PER_G, TOK_PER_G)]],
                vals_v.at[pl.ds(g * TOK_PER_G, TOK_PER_G)], sem_g))

    lanes = lax.iota(jnp.int32, LANES)
    tail_mask = lanes < TAIL
    lane_base = lanes * SENT_PER_W

    def sentence(s, carry):
        base = s * L
        for k in range(CHUNKS):
            tok = toks_v[pl.ds(base + k * LANES, LANES)]
            marker = lanes + (s * 256 + k * LANES)
            if k == CHUNKS - 1:
                plsc.store_scatter(stamp_v, [tok], marker, mask=tail_mask)
            else:
                plsc.store_scatter(stamp_v, [tok], marker)
        acc = jnp.zeros((LANES,), jnp.float32)
        for k in range(CHUNKS):
            tok = toks_v[pl.ds(base + k * LANES, LANES)]
            val = vals_v[pl.ds(base + k * LANES, LANES)]
            if k == CHUNKS - 1:
                back = plsc.load_gather(stamp_v, [tok], mask=tail_mask)
            else:
                back = plsc.load_gather(stamp_v, [tok])
            marker = lanes + (s * 256 + k * LANES)
            keep = (back == marker) & (tok != PAD)
            if k == CHUNKS - 1:
                keep = keep & tail_mask
            acc = acc + jnp.where(keep, val, 0.0)
        plsc.store_scatter(scoret_v, [lane_base + s], acc)
        return carry

    for g in range(GSPLIT):
        with jax.named_scope("gather_wait"):
            gathers[g].wait()
        with jax.named_scope("dedup_compute"):
            lax.fori_loop(g * SENT_PER_G, (g + 1) * SENT_PER_G, sentence, 0)

    with jax.named_scope("score_fold"):
        half0 = jnp.zeros((LANES,), jnp.float32)
        half1 = jnp.zeros((LANES,), jnp.float32)
        for r in range(LANES):
            half0 = half0 + scoret_v[pl.ds(r * SENT_PER_W, LANES)]
            half1 = half1 + scoret_v[pl.ds(r * SENT_PER_W + LANES, LANES)]
        score_v[pl.ds(0, LANES)] = half0
        score_v[pl.ds(LANES, LANES)] = half1

    with jax.named_scope("store_scores"):
        pltpu.sync_copy(score_v, out_hbm.at[pl.ds(wid * SENT_PER_W, SENT_PER_W)])


_nb_kernel = functools.partial(
    pl.kernel,
    out_type=jax.ShapeDtypeStruct((B,), jnp.float32),
    mesh=plsc.VectorSubcoreMesh(core_axis_name="c", subcore_axis_name="s"),
    compiler_params=pltpu.CompilerParams(needs_layout_passes=False),
    scratch_types=[
        pltpu.VMEM((TOK_PER_W + LANES,), jnp.int32),
        pltpu.VMEM((TOK_PER_W + LANES,), jnp.float32),
        pltpu.VMEM((VOCAB,), jnp.int32),
        pltpu.VMEM((SENT_PER_W,), jnp.float32),
        pltpu.VMEM((LANES * SENT_PER_W,), jnp.float32),
        pltpu.VMEM_SHARED((VOCAB,), jnp.float32),
        pltpu.SemaphoreType.DMA,
        pltpu.SemaphoreType.DMA,
    ],
)(_nb_body)


@jax.jit
def kernel(sentences, log_count_ratio, bias):
    toks = sentences.T.reshape(B * L)
    scores = _nb_kernel(toks, log_count_ratio) + bias
    return jnp.stack([-scores, scores], axis=1)
